# baseline (speedup 1.0000x reference)
import functools

import jax
import jax.numpy as jnp
from jax import lax
from jax.experimental import pallas as pl
from jax.experimental.pallas import tpu as pltpu
from jax.experimental.pallas import tpu_sc as plsc

_B, _D, _H, _W = 16, 64, 32, 32
_K = 1024
_HW = _H * _W
_N = _B * _HW


def _vq_tc(x_ref, emb_ref, idx_ref, loss_ref):
    xb = x_ref[0]
    emb = emb_ref[...]
    flat = xb.T
    xnT = jnp.sum(flat * flat, axis=1, keepdims=True).T
    enT = jnp.sum(emb * emb, axis=1)[:, None]
    prodT = lax.dot_general(emb, xb, (((1,), (0,)), ((), ())),
                            preferred_element_type=jnp.float32)
    d2 = xnT - 2.0 * prodT + enT
    dist = jnp.sqrt(jnp.maximum(d2, 0.0))
    m = jnp.min(dist, axis=0)
    ks = lax.broadcasted_iota(jnp.int32, d2.shape, 0)
    idx = jnp.min(jnp.where(dist == m[None, :], ks, _K), axis=0)
    idx_ref[0, 0, :] = idx
    md = m * m

    @pl.when(pl.program_id(0) == 0)
    def _init():
        loss_ref[...] = jnp.zeros((1, 1), jnp.float32)

    loss_ref[...] += jnp.full((1, 1), jnp.sum(md), jnp.float32)


_NC, _NS = 2, 16
_NW = _NC * _NS
_CHUNKS = 2
_BC = _B // _CHUNKS
_NH = _BC * _HW
_BPW = _NH // _NW


_DP = 128


@functools.cache
def _make_sc_gather():
    @functools.partial(
        pl.kernel,
        mesh=plsc.VectorSubcoreMesh(core_axis_name="c", subcore_axis_name="s"),
        out_type=jax.ShapeDtypeStruct((_NH, _DP), jnp.float32),
        scratch_types=[
            pltpu.VMEM((_BPW,), jnp.int32),
            pltpu.VMEM((_BPW, _DP), jnp.float32),
            pltpu.SemaphoreType.DMA,
        ],
    )
    def _sc_gather(table_hbm, idx_hbm, out_hbm, idx_v, rows_v, sem):
        wid = lax.axis_index("s") * _NC + lax.axis_index("c")
        base = wid * _BPW
        pltpu.sync_copy(idx_hbm.at[pl.ds(base, _BPW)], idx_v)
        pltpu.async_copy(table_hbm.at[idx_v], rows_v, sem).wait()
        pltpu.sync_copy(rows_v, out_hbm.at[pl.ds(base, _BPW)])

    return _sc_gather


def kernel(x, embeddings):
    x3 = x.reshape(_B, _D, _HW)
    emb_pad = jnp.pad(embeddings, ((0, 0), (0, _DP - _D)))
    sc_gather = _make_sc_gather()
    tc = functools.partial(
        pl.pallas_call,
        _vq_tc,
        grid=(_BC,),
        in_specs=[
            pl.BlockSpec((1, _D, _HW), lambda i: (i, 0, 0)),
            pl.BlockSpec((_K, _D), lambda i: (0, 0)),
        ],
        out_specs=[
            pl.BlockSpec((1, 1, _HW), lambda i: (i, 0, 0)),
            pl.BlockSpec((1, 1), lambda i: (0, 0)),
        ],
        out_shape=[
            jax.ShapeDtypeStruct((_BC, 1, _HW), jnp.int32),
            jax.ShapeDtypeStruct((1, 1), jnp.float32),
        ],
    )
    outs, idxs, loss_sum = [], [], 0.0
    for c in range(_CHUNKS):
        xc = lax.slice_in_dim(x3, c * _BC, (c + 1) * _BC, axis=0)
        idx3c, loss_c = tc()(xc, embeddings)
        qc = sc_gather(emb_pad, idx3c.reshape(_NH))
        outs.append(qc[:, :_D].reshape(_BC, _H, _W, _D).transpose(0, 3, 1, 2))
        idxs.append(idx3c.reshape(_BC, _H, _W))
        loss_sum = loss_sum + loss_c[0, 0]
    out = jnp.concatenate(outs, axis=0)
    idx = jnp.concatenate(idxs, axis=0)
    loss = loss_sum / (_N * _D)
    return out, idx, loss, loss


# speedup vs baseline: 1.1800x; 1.1800x over previous
import functools

import jax
import jax.numpy as jnp
from jax import lax
from jax.experimental import pallas as pl
from jax.experimental.pallas import tpu as pltpu
from jax.experimental.pallas import tpu_sc as plsc

_B, _D, _H, _W = 16, 64, 32, 32
_K = 1024
_HW = _H * _W
_N = _B * _HW


def _vq_tc(x_ref, emb_ref, idx_ref, loss_ref):
    xb = x_ref[0]
    emb = emb_ref[...]
    flat = xb.T
    xnT = jnp.sum(flat * flat, axis=1, keepdims=True).T
    enT = jnp.sum(emb * emb, axis=1)[:, None]
    prod2T = lax.dot_general(emb * -2.0, xb, (((1,), (0,)), ((), ())),
                             preferred_element_type=jnp.float32)
    d2 = xnT + prod2T + enT
    dist = jnp.sqrt(jnp.maximum(d2, 0.0))
    m = jnp.min(dist, axis=0)
    ks = lax.broadcasted_iota(jnp.int32, (_K, 1), 0)
    idx = jnp.min(jnp.where(dist == m[None, :], ks, _K), axis=0)
    idx_ref[0, 0, :] = idx
    md = m * m

    @pl.when(pl.program_id(0) == 0)
    def _init():
        loss_ref[...] = jnp.zeros((1, 1), jnp.float32)

    loss_ref[...] += jnp.full((1, 1), jnp.sum(md), jnp.float32)


_NC, _NS = 2, 16
_NW = _NC * _NS
_BPW = _N // _NW


_DP = 128


@functools.cache
def _make_sc_gather():
    @functools.partial(
        pl.kernel,
        mesh=plsc.VectorSubcoreMesh(core_axis_name="c", subcore_axis_name="s"),
        out_type=jax.ShapeDtypeStruct((_N, _DP), jnp.float32),
        scratch_types=[
            pltpu.VMEM((_BPW,), jnp.int32),
            pltpu.VMEM((_BPW, _DP), jnp.float32),
            pltpu.SemaphoreType.DMA,
        ],
    )
    def _sc_gather(table_hbm, idx_hbm, out_hbm, idx_v, rows_v, sem):
        wid = lax.axis_index("s") * _NC + lax.axis_index("c")
        base = wid * _BPW
        pltpu.sync_copy(idx_hbm.at[pl.ds(base, _BPW)], idx_v)
        pltpu.async_copy(table_hbm.at[idx_v], rows_v, sem).wait()
        pltpu.sync_copy(rows_v, out_hbm.at[pl.ds(base, _BPW)])

    return _sc_gather


def kernel(x, embeddings):
    x3 = x.reshape(_B, _D, _HW)
    idx3, loss_sum = pl.pallas_call(
        _vq_tc,
        grid=(_B,),
        in_specs=[
            pl.BlockSpec((1, _D, _HW), lambda i: (i, 0, 0)),
            pl.BlockSpec((_K, _D), lambda i: (0, 0)),
        ],
        out_specs=[
            pl.BlockSpec((1, 1, _HW), lambda i: (i, 0, 0)),
            pl.BlockSpec((1, 1), lambda i: (0, 0)),
        ],
        out_shape=[
            jax.ShapeDtypeStruct((_B, 1, _HW), jnp.int32),
            jax.ShapeDtypeStruct((1, 1), jnp.float32),
        ],
    )(x3, embeddings)
    flat_idx = idx3.reshape(_N)
    emb_pad = jnp.pad(embeddings, ((0, 0), (0, _DP - _D)))
    q = _make_sc_gather()(emb_pad, flat_idx)
    out = q[:, :_D].reshape(_B, _H, _W, _D).transpose(0, 3, 1, 2)
    idx = idx3.reshape(_B, _H, _W)
    loss = loss_sum[0, 0] / (_N * _D)
    return out, idx, loss, loss
